# baseline (speedup 1.0000x reference)
import functools

import jax
import jax.numpy as jnp
from jax import lax
from jax.experimental import pallas as pl
from jax.experimental.pallas import tpu as pltpu
from jax.experimental.pallas import tpu_sc as plsc

N = 10000
E = 320000
F_IN = 128
H = 256
FH = 128
C = 10
L = 3
G = 128

NC = 2
NS = 16

_CHUNK = 80
_NB = 256
_NBUF = 4
_NIDX = 8
_EPT = _NB * _CHUNK
_E_PAD = NS * _EPT
_ROWS_PER_TILE = 624
_ROWS_TAIL = N - NS * _ROWS_PER_TILE

_BM = 1000
_GRID = N // _BM


_NW = NC * NS
_M = _E_PAD // _NW
_KB = 1280
_LANES = 16


def _sc_sort_body(src_hbm, dst_hbm, osrc_hbm, odst_hbm, src_v, dst_v, out_s,
                  out_d, off2):
    c = lax.axis_index("c")
    s = lax.axis_index("s")
    w = s * NC + c
    pltpu.sync_copy(src_hbm.at[w], src_v)
    pltpu.sync_copy(dst_hbm.at[w], dst_v)
    lanes = lax.iota(jnp.int32, _LANES)
    ones = jnp.ones((_LANES,), jnp.int32)

    def zero(b, _):
        off2[pl.ds(b * _LANES, _LANES)] = jnp.zeros((_LANES,), jnp.int32)
        return 0

    lax.fori_loop(0, _KB, zero, 0)

    def hist(i, _):
        v = src_v[pl.ds(i * _LANES, _LANES)]
        i16 = jnp.right_shift(v, 3) * _LANES + lanes
        base = plsc.load_gather(off2, [i16])
        plsc.store_scatter(off2, [i16], base + ones)
        return 0

    lax.fori_loop(0, _M // _LANES, hist, 0)

    def excl(b, carry):
        hv = off2[pl.ds(b * _LANES, _LANES)]
        inc = plsc.cumsum(hv)
        off2[pl.ds(b * _LANES, _LANES)] = carry + inc - hv
        return carry + jnp.sum(hv)

    lax.fori_loop(0, _KB, excl, jnp.int32(0))

    def scat(i, _):
        sv = src_v[pl.ds(i * _LANES, _LANES)]
        dv = dst_v[pl.ds(i * _LANES, _LANES)]
        i16 = jnp.right_shift(sv, 3) * _LANES + lanes
        base = plsc.load_gather(off2, [i16])
        plsc.store_scatter(off2, [i16], base + 1)
        plsc.store_scatter(out_s, [base], sv)
        plsc.store_scatter(out_d, [base], dv)
        return 0

    lax.fori_loop(0, _M // _LANES, scat, 0)
    pltpu.sync_copy(out_s, osrc_hbm.at[w])
    pltpu.sync_copy(out_d, odst_hbm.at[w])


def _sc_sort(src2, dst2):
    mesh = plsc.VectorSubcoreMesh(core_axis_name="c", subcore_axis_name="s")
    f = pl.kernel(
        _sc_sort_body,
        out_type=[
            jax.ShapeDtypeStruct((_NW, _M), jnp.int32),
            jax.ShapeDtypeStruct((_NW, _M), jnp.int32),
        ],
        mesh=mesh,
        scratch_types=(
            [pltpu.VMEM((_M,), jnp.int32)] * 4
            + [pltpu.VMEM((_KB * _LANES,), jnp.int32)]
        ),
        compiler_params=pltpu.CompilerParams(needs_layout_passes=False),
    )
    return f(src2, dst2)

def _sc_agg_body(h2_hbm, src_hbm, dst_hbm, out_hbm, *scr):
    rows = list(scr[0:_NBUF])
    srcb = list(scr[_NBUF:_NBUF + _NIDX])
    dstb = list(scr[_NBUF + _NIDX:_NBUF + 2 * _NIDX])
    gsem = list(scr[_NBUF + 2 * _NIDX:2 * _NBUF + 2 * _NIDX])
    ssem = list(scr[2 * _NBUF + 2 * _NIDX:2 * _NBUF + 3 * _NIDX])
    dsem = list(scr[2 * _NBUF + 3 * _NIDX:2 * _NBUF + 4 * _NIDX])
    agg_sh = scr[-1]
    c = lax.axis_index("c")
    s = lax.axis_index("s")
    rbase = s * _ROWS_PER_TILE
    pltpu.sync_copy(h2_hbm.at[pl.ds(c * N + rbase, _ROWS_PER_TILE)],
                    agg_sh.at[pl.ds(rbase, _ROWS_PER_TILE)])

    @pl.when(s == 0)
    def _():
        tb = NS * _ROWS_PER_TILE
        pltpu.sync_copy(h2_hbm.at[pl.ds(c * N + tb, _ROWS_TAIL)],
                        agg_sh.at[pl.ds(tb, _ROWS_TAIL)])

    rot = (s * NC + c) * 5

    def rc(i):
        return lax.rem(i + rot, _NB)

    for k in range(_NIDX):
        pltpu.async_copy(src_hbm.at[c, s, rc(k)], srcb[k], ssem[k])
        pltpu.async_copy(dst_hbm.at[s, rc(k)], dstb[k], dsem[k])
    for b in range(_NBUF):
        pltpu.make_async_copy(src_hbm.at[c, s, rc(b)], srcb[b],
                              ssem[b]).wait()
        pltpu.async_copy(h2_hbm.at[srcb[b]], rows[b], gsem[b])

    plsc.subcore_barrier()

    def body(g, _):
        for ib in range(_NIDX):
            i = g * _NIDX + ib
            b = ib % _NBUF
            pltpu.make_async_copy(h2_hbm.at[srcb[ib]], rows[b],
                                  gsem[b]).wait()
            pltpu.make_async_copy(dst_hbm.at[s, rc(i)], dstb[ib],
                                  dsem[ib]).wait()
            pltpu.sync_copy(rows[b], agg_sh.at[dstb[ib]], add=True)

            i2 = (ib + _NBUF) % _NIDX

            @pl.when(i + _NBUF < _NB)
            def _():
                pltpu.make_async_copy(src_hbm.at[c, s, rc(i + _NBUF)],
                                      srcb[i2], ssem[i2]).wait()
                pltpu.async_copy(h2_hbm.at[srcb[i2]], rows[b], gsem[b])

            @pl.when(i + _NIDX < _NB)
            def _():
                pltpu.async_copy(src_hbm.at[c, s, rc(i + _NIDX)], srcb[ib],
                                 ssem[ib])
                pltpu.async_copy(dst_hbm.at[s, rc(i + _NIDX)], dstb[ib],
                                 dsem[ib])
        return 0

    lax.fori_loop(0, _NB // _NIDX, body, 0)
    plsc.subcore_barrier()
    pltpu.sync_copy(agg_sh.at[pl.ds(rbase, _ROWS_PER_TILE)],
                    out_hbm.at[c, pl.ds(rbase, _ROWS_PER_TILE)])

    @pl.when(s == 0)
    def _():
        tb = NS * _ROWS_PER_TILE
        pltpu.sync_copy(agg_sh.at[pl.ds(tb, _ROWS_TAIL)],
                        out_hbm.at[c, pl.ds(tb, _ROWS_TAIL)])


def _sc_aggregate(h2, srcs, dst4):
    mesh = plsc.VectorSubcoreMesh(core_axis_name="c", subcore_axis_name="s")
    f = pl.kernel(
        _sc_agg_body,
        out_type=jax.ShapeDtypeStruct((NC, N, FH), jnp.float32),
        mesh=mesh,
        scratch_types=(
            [pltpu.VMEM((_CHUNK, FH), jnp.float32)] * _NBUF
            + [pltpu.VMEM((_CHUNK,), jnp.int32)] * (2 * _NIDX)
            + [pltpu.SemaphoreType.DMA] * (_NBUF + 2 * _NIDX)
            + [pltpu.VMEM_SHARED((N + 8, FH), jnp.float32)]
        ),
    )
    return f(h2, srcs, dst4)


def _stats_x_body(x_ref, st_ref):
    i = pl.program_id(0)

    @pl.when(i == 0)
    def _():
        st_ref[...] = jnp.zeros_like(st_ref)

    x = x_ref[...]
    s1 = jnp.sum(x, axis=0)
    s2 = jnp.sum(x * x, axis=0)
    st_ref[...] += jnp.concatenate(
        [s1[None], s2[None], jnp.zeros((6, x.shape[1]), jnp.float32)], axis=0)


def _stats_x(x):
    fw = x.shape[1]
    return pl.pallas_call(
        _stats_x_body,
        grid=(_GRID,),
        in_specs=[pl.BlockSpec((_BM, fw), lambda i: (i, 0))],
        out_specs=pl.BlockSpec((8, fw), lambda i: (0, 0)),
        out_shape=jax.ShapeDtypeStruct((8, fw), jnp.float32),
    )(x)


def _bn0_mm_body(x_ref, s_ref, o_ref, w_ref, b_ref, out_ref):
    x = x_ref[...] * s_ref[...] + o_ref[...]
    h = jnp.maximum(
        jnp.dot(x, w_ref[...], preferred_element_type=jnp.float32)
        + b_ref[...], 0.0)
    out_ref[0] = h[:, :FH]
    out_ref[1] = h[:, FH:]


def _bn0_mm(x, s, o, w, b):
    return pl.pallas_call(
        _bn0_mm_body,
        grid=(_GRID,),
        in_specs=[
            pl.BlockSpec((_BM, F_IN), lambda i: (i, 0)),
            pl.BlockSpec((1, F_IN), lambda i: (0, 0)),
            pl.BlockSpec((1, F_IN), lambda i: (0, 0)),
            pl.BlockSpec((F_IN, H), lambda i: (0, 0)),
            pl.BlockSpec((1, H), lambda i: (0, 0)),
        ],
        out_specs=pl.BlockSpec((NC, _BM, FH), lambda i: (0, i, 0)),
        out_shape=jax.ShapeDtypeStruct((NC, N, FH), jnp.float32),
    )(x, s, o, w, b)


def _mm1_stats_body(z_ref, w_ref, b_ref, t_ref, st_ref):
    i = pl.program_id(0)
    z = jnp.concatenate([z_ref[0], z_ref[1]], axis=1)
    t = jnp.dot(z, w_ref[...], preferred_element_type=jnp.float32) + b_ref[...]
    t_ref[...] = t

    @pl.when(i == 0)
    def _():
        st_ref[...] = jnp.zeros_like(st_ref)

    s1 = jnp.sum(t, axis=0)
    s2 = jnp.sum(t * t, axis=0)
    st_ref[...] += jnp.concatenate(
        [s1[None], s2[None], jnp.zeros((6, H), jnp.float32)], axis=0)


def _mm1_stats(z2, w, b):
    return pl.pallas_call(
        _mm1_stats_body,
        grid=(_GRID,),
        in_specs=[
            pl.BlockSpec((NC, _BM, FH), lambda i: (0, i, 0)),
            pl.BlockSpec((H, H), lambda i: (0, 0)),
            pl.BlockSpec((1, H), lambda i: (0, 0)),
        ],
        out_specs=[
            pl.BlockSpec((_BM, H), lambda i: (i, 0)),
            pl.BlockSpec((8, H), lambda i: (0, 0)),
        ],
        out_shape=[
            jax.ShapeDtypeStruct((N, H), jnp.float32),
            jax.ShapeDtypeStruct((8, H), jnp.float32),
        ],
    )(z2, w, b)


def _bn_mm2_body(split_out, t_ref, s_ref, o_ref, w_ref, b_ref, out_ref):
    t = jnp.maximum(t_ref[...] * s_ref[...] + o_ref[...], 0.0)
    h = jnp.maximum(
        jnp.dot(t, w_ref[...], preferred_element_type=jnp.float32)
        + b_ref[...], 0.0)
    if split_out:
        out_ref[0] = h[:, :FH]
        out_ref[1] = h[:, FH:]
    else:
        out_ref[...] = h


def _bn_mm2(t, s, o, w, b, split_out):
    if split_out:
        out_spec = pl.BlockSpec((NC, _BM, FH), lambda i: (0, i, 0))
        out_shape = jax.ShapeDtypeStruct((NC, N, FH), jnp.float32)
    else:
        out_spec = pl.BlockSpec((_BM, H), lambda i: (i, 0))
        out_shape = jax.ShapeDtypeStruct((N, H), jnp.float32)
    return pl.pallas_call(
        functools.partial(_bn_mm2_body, split_out),
        grid=(_GRID,),
        in_specs=[
            pl.BlockSpec((_BM, H), lambda i: (i, 0)),
            pl.BlockSpec((1, H), lambda i: (0, 0)),
            pl.BlockSpec((1, H), lambda i: (0, 0)),
            pl.BlockSpec((H, H), lambda i: (0, 0)),
            pl.BlockSpec((1, H), lambda i: (0, 0)),
        ],
        out_specs=out_spec,
        out_shape=out_shape,
    )(t, s, o, w, b)


def _pool_head_body(h_ref, batch_ref, fcg_ref, fcb_ref, lw_ref, lb_ref,
                    hdg_ref, hdb_ref, cw_ref, cb_ref, hg_ref, out_ref):
    i = pl.program_id(0)

    @pl.when(i == 0)
    def _():
        hg_ref[...] = jnp.zeros_like(hg_ref)
        out_ref[...] = jnp.zeros_like(out_ref)

    bv = batch_ref[...].reshape(1, _BM)
    gi = lax.broadcasted_iota(jnp.int32, (G, _BM), 0)
    onehot = (gi == bv).astype(jnp.float32)
    hg_ref[...] += jnp.dot(onehot, h_ref[...],
                           preferred_element_type=jnp.float32)

    @pl.when(i == _GRID - 1)
    def _():
        hg = hg_ref[...]
        m = jnp.mean(hg, axis=0)
        v = jnp.mean(hg * hg, axis=0) - m * m
        hgn = (hg - m) * lax.rsqrt(v + 1e-5) * fcg_ref[...] + fcb_ref[...]
        h1 = jnp.maximum(
            jnp.dot(hgn, lw_ref[...], preferred_element_type=jnp.float32)
            + lb_ref[...], 0.0)
        m2 = jnp.mean(h1, axis=0)
        v2 = jnp.mean(h1 * h1, axis=0) - m2 * m2
        h1n = (h1 - m2) * lax.rsqrt(v2 + 1e-5) * hdg_ref[...] + hdb_ref[...]
        logits = jnp.dot(h1n, cw_ref[...],
                         preferred_element_type=jnp.float32) + cb_ref[...]
        valid = lax.broadcasted_iota(jnp.int32, (G, 128), 1) < C
        neg = jnp.float32(-1e30)
        lmax = jnp.max(jnp.where(valid, logits, neg), axis=1, keepdims=True)
        shifted = logits - lmax
        se = jnp.sum(jnp.where(valid, jnp.exp(shifted), 0.0), axis=1,
                     keepdims=True)
        out = shifted - jnp.log(se)
        out_ref[...] = jnp.where(valid, out, 0.0)


def _pool_head(h, batch3, fcg, fcb, lw, lb, hdg, hdb, cwp, cbp):
    return pl.pallas_call(
        _pool_head_body,
        grid=(_GRID,),
        in_specs=[
            pl.BlockSpec((_BM, H), lambda i: (i, 0)),
            pl.BlockSpec((1, 1, _BM), lambda i: (i, 0, 0)),
            pl.BlockSpec((1, H), lambda i: (0, 0)),
            pl.BlockSpec((1, H), lambda i: (0, 0)),
            pl.BlockSpec((H, H), lambda i: (0, 0)),
            pl.BlockSpec((1, H), lambda i: (0, 0)),
            pl.BlockSpec((1, H), lambda i: (0, 0)),
            pl.BlockSpec((1, H), lambda i: (0, 0)),
            pl.BlockSpec((H, 128), lambda i: (0, 0)),
            pl.BlockSpec((1, 128), lambda i: (0, 0)),
        ],
        out_specs=[
            pl.BlockSpec((G, H), lambda i: (0, 0)),
            pl.BlockSpec((G, 128), lambda i: (0, 0)),
        ],
        out_shape=[
            jax.ShapeDtypeStruct((G, H), jnp.float32),
            jax.ShapeDtypeStruct((G, 128), jnp.float32),
        ],
    )(h, batch3, fcg, fcb, lw, lb, hdg, hdb, cwp, cbp)


def _scale_offset(st, g, b, n):
    m = st[0] / n
    v = st[1] / n - m * m
    s = g * lax.rsqrt(v + 1e-5)
    o = b - m * s
    return s[None, :], o[None, :]


def kernel(x, edge_index, batch, bn_feat_g, bn_feat_b, W0, b0, gin_W1, gin_b1,
           gin_bn_g, gin_bn_b, gin_W2, gin_b2, bn_fc_g, bn_fc_b, lin_W, lin_b,
           bn_hid_g, bn_hid_b, cls_W, cls_b):
    src = edge_index[0]
    dst = edge_index[1]
    pad = _E_PAD - E
    srcp = jnp.concatenate([src, jnp.zeros((pad,), jnp.int32)])
    dstp = jnp.concatenate([dst, jnp.full((pad,), N, jnp.int32)])
    srcs = jnp.stack([srcp, srcp + N]).reshape(NC, NS, _NB, _CHUNK)
    dst4 = dstp.reshape(NS, _NB, _CHUNK)
    batch3 = batch.reshape(_GRID, 1, _BM)

    st0 = _stats_x(x)
    s0, o0 = _scale_offset(st0, bn_feat_g, bn_feat_b, N)
    h2 = _bn0_mm(x, s0, o0, W0, b0[None, :])

    for i in range(L):
        z2 = _sc_aggregate(h2.reshape(NC * N, FH), srcs, dst4)
        t, st = _mm1_stats(z2, gin_W1[i], gin_b1[i][None, :])
        si, oi = _scale_offset(st, gin_bn_g[i], gin_bn_b[i], N)
        last = i == L - 1
        h2 = _bn_mm2(t, si, oi, gin_W2[i], gin_b2[i][None, :],
                     split_out=not last)

    cwp = jnp.zeros((H, 128), jnp.float32).at[:, :C].set(cls_W)
    cbp = jnp.zeros((1, 128), jnp.float32).at[0, :C].set(cls_b)
    _, logits = _pool_head(h2, batch3, bn_fc_g[None, :], bn_fc_b[None, :],
                           lin_W, lin_b[None, :], bn_hid_g[None, :],
                           bn_hid_b[None, :], cwp, cbp)
    return logits[:, :C]


# speedup vs baseline: 1.8356x; 1.8356x over previous
import functools

import jax
import jax.numpy as jnp
from jax import lax
from jax.experimental import pallas as pl
from jax.experimental.pallas import tpu as pltpu
from jax.experimental.pallas import tpu_sc as plsc

N = 10000
E = 320000
F_IN = 128
H = 256
FH = 128
C = 10
L = 3
G = 128

NC = 2
NS = 16

_CHUNK = 96
_NB = 210
_NBUF = 3
_NIDX = 6
_EPT = _NB * _CHUNK
_E_PAD = NS * _EPT
_ROWS_PER_TILE = 624
_ROWS_TAIL = N - NS * _ROWS_PER_TILE

_BM = 1000
_GRID = N // _BM


def _sc_agg_body(h2_hbm, src_hbm, dst_hbm, out_hbm, *scr):
    rows = list(scr[0:_NBUF])
    srcb = list(scr[_NBUF:_NBUF + _NIDX])
    dstb = list(scr[_NBUF + _NIDX:_NBUF + 2 * _NIDX])
    gsem = list(scr[_NBUF + 2 * _NIDX:2 * _NBUF + 2 * _NIDX])
    ssem = list(scr[2 * _NBUF + 2 * _NIDX:2 * _NBUF + 3 * _NIDX])
    dsem = list(scr[2 * _NBUF + 3 * _NIDX:2 * _NBUF + 4 * _NIDX])
    agg_sh = scr[-1]
    c = lax.axis_index("c")
    s = lax.axis_index("s")
    rbase = s * _ROWS_PER_TILE
    pltpu.sync_copy(h2_hbm.at[pl.ds(c * N + rbase, _ROWS_PER_TILE)],
                    agg_sh.at[pl.ds(rbase, _ROWS_PER_TILE)])

    @pl.when(s == 0)
    def _():
        tb = NS * _ROWS_PER_TILE
        pltpu.sync_copy(h2_hbm.at[pl.ds(c * N + tb, _ROWS_TAIL)],
                        agg_sh.at[pl.ds(tb, _ROWS_TAIL)])

    for k in range(_NIDX):
        pltpu.async_copy(src_hbm.at[c, s, k], srcb[k], ssem[k])
        pltpu.async_copy(dst_hbm.at[s, k], dstb[k], dsem[k])
    for b in range(_NBUF):
        pltpu.make_async_copy(src_hbm.at[c, s, b], srcb[b],
                              ssem[b]).wait()
        pltpu.async_copy(h2_hbm.at[srcb[b]], rows[b], gsem[b])

    plsc.subcore_barrier()

    def body(g, _):
        for ib in range(_NIDX):
            i = g * _NIDX + ib
            b = ib % _NBUF
            pltpu.make_async_copy(h2_hbm.at[srcb[ib]], rows[b],
                                  gsem[b]).wait()
            pltpu.make_async_copy(dst_hbm.at[s, i], dstb[ib],
                                  dsem[ib]).wait()
            pltpu.sync_copy(rows[b], agg_sh.at[dstb[ib]], add=True)

            i2 = (ib + _NBUF) % _NIDX

            @pl.when(i + _NBUF < _NB)
            def _():
                pltpu.make_async_copy(src_hbm.at[c, s, i + _NBUF],
                                      srcb[i2], ssem[i2]).wait()
                pltpu.async_copy(h2_hbm.at[srcb[i2]], rows[b], gsem[b])

            @pl.when(i + _NIDX < _NB)
            def _():
                pltpu.async_copy(src_hbm.at[c, s, i + _NIDX], srcb[ib],
                                 ssem[ib])
                pltpu.async_copy(dst_hbm.at[s, i + _NIDX], dstb[ib],
                                 dsem[ib])
        return 0

    lax.fori_loop(0, _NB // _NIDX, body, 0)
    plsc.subcore_barrier()
    pltpu.sync_copy(agg_sh.at[pl.ds(rbase, _ROWS_PER_TILE)],
                    out_hbm.at[c, pl.ds(rbase, _ROWS_PER_TILE)])

    @pl.when(s == 0)
    def _():
        tb = NS * _ROWS_PER_TILE
        pltpu.sync_copy(agg_sh.at[pl.ds(tb, _ROWS_TAIL)],
                        out_hbm.at[c, pl.ds(tb, _ROWS_TAIL)])


def _sc_aggregate(h2, srcs, dst4):
    mesh = plsc.VectorSubcoreMesh(core_axis_name="c", subcore_axis_name="s")
    f = pl.kernel(
        _sc_agg_body,
        out_type=jax.ShapeDtypeStruct((NC, N, FH), jnp.float32),
        mesh=mesh,
        scratch_types=(
            [pltpu.VMEM((_CHUNK, FH), jnp.float32)] * _NBUF
            + [pltpu.VMEM((_CHUNK,), jnp.int32)] * (2 * _NIDX)
            + [pltpu.SemaphoreType.DMA] * (_NBUF + 2 * _NIDX)
            + [pltpu.VMEM_SHARED((N + 8, FH), jnp.float32)]
        ),
    )
    return f(h2, srcs, dst4)


def _stats_x_body(x_ref, st_ref):
    i = pl.program_id(0)

    @pl.when(i == 0)
    def _():
        st_ref[...] = jnp.zeros_like(st_ref)

    x = x_ref[...]
    s1 = jnp.sum(x, axis=0)
    s2 = jnp.sum(x * x, axis=0)
    st_ref[...] += jnp.concatenate(
        [s1[None], s2[None], jnp.zeros((6, x.shape[1]), jnp.float32)], axis=0)


def _stats_x(x):
    fw = x.shape[1]
    return pl.pallas_call(
        _stats_x_body,
        grid=(_GRID,),
        in_specs=[pl.BlockSpec((_BM, fw), lambda i: (i, 0))],
        out_specs=pl.BlockSpec((8, fw), lambda i: (0, 0)),
        out_shape=jax.ShapeDtypeStruct((8, fw), jnp.float32),
    )(x)


def _bn0_mm_body(x_ref, s_ref, o_ref, w_ref, b_ref, out_ref):
    x = x_ref[...] * s_ref[...] + o_ref[...]
    h = jnp.maximum(
        jnp.dot(x, w_ref[...], preferred_element_type=jnp.float32)
        + b_ref[...], 0.0)
    out_ref[0] = h[:, :FH]
    out_ref[1] = h[:, FH:]


def _bn0_mm(x, s, o, w, b):
    return pl.pallas_call(
        _bn0_mm_body,
        grid=(_GRID,),
        in_specs=[
            pl.BlockSpec((_BM, F_IN), lambda i: (i, 0)),
            pl.BlockSpec((1, F_IN), lambda i: (0, 0)),
            pl.BlockSpec((1, F_IN), lambda i: (0, 0)),
            pl.BlockSpec((F_IN, H), lambda i: (0, 0)),
            pl.BlockSpec((1, H), lambda i: (0, 0)),
        ],
        out_specs=pl.BlockSpec((NC, _BM, FH), lambda i: (0, i, 0)),
        out_shape=jax.ShapeDtypeStruct((NC, N, FH), jnp.float32),
    )(x, s, o, w, b)


def _mm1_stats_body(z_ref, w_ref, b_ref, t_ref, st_ref):
    i = pl.program_id(0)
    z = jnp.concatenate([z_ref[0], z_ref[1]], axis=1)
    t = jnp.dot(z, w_ref[...], preferred_element_type=jnp.float32) + b_ref[...]
    t_ref[...] = t

    @pl.when(i == 0)
    def _():
        st_ref[...] = jnp.zeros_like(st_ref)

    s1 = jnp.sum(t, axis=0)
    s2 = jnp.sum(t * t, axis=0)
    st_ref[...] += jnp.concatenate(
        [s1[None], s2[None], jnp.zeros((6, H), jnp.float32)], axis=0)


def _mm1_stats(z2, w, b):
    return pl.pallas_call(
        _mm1_stats_body,
        grid=(_GRID,),
        in_specs=[
            pl.BlockSpec((NC, _BM, FH), lambda i: (0, i, 0)),
            pl.BlockSpec((H, H), lambda i: (0, 0)),
            pl.BlockSpec((1, H), lambda i: (0, 0)),
        ],
        out_specs=[
            pl.BlockSpec((_BM, H), lambda i: (i, 0)),
            pl.BlockSpec((8, H), lambda i: (0, 0)),
        ],
        out_shape=[
            jax.ShapeDtypeStruct((N, H), jnp.float32),
            jax.ShapeDtypeStruct((8, H), jnp.float32),
        ],
    )(z2, w, b)


def _bn_mm2_body(split_out, t_ref, s_ref, o_ref, w_ref, b_ref, out_ref):
    t = jnp.maximum(t_ref[...] * s_ref[...] + o_ref[...], 0.0)
    h = jnp.maximum(
        jnp.dot(t, w_ref[...], preferred_element_type=jnp.float32)
        + b_ref[...], 0.0)
    if split_out:
        out_ref[0] = h[:, :FH]
        out_ref[1] = h[:, FH:]
    else:
        out_ref[...] = h


def _bn_mm2(t, s, o, w, b, split_out):
    if split_out:
        out_spec = pl.BlockSpec((NC, _BM, FH), lambda i: (0, i, 0))
        out_shape = jax.ShapeDtypeStruct((NC, N, FH), jnp.float32)
    else:
        out_spec = pl.BlockSpec((_BM, H), lambda i: (i, 0))
        out_shape = jax.ShapeDtypeStruct((N, H), jnp.float32)
    return pl.pallas_call(
        functools.partial(_bn_mm2_body, split_out),
        grid=(_GRID,),
        in_specs=[
            pl.BlockSpec((_BM, H), lambda i: (i, 0)),
            pl.BlockSpec((1, H), lambda i: (0, 0)),
            pl.BlockSpec((1, H), lambda i: (0, 0)),
            pl.BlockSpec((H, H), lambda i: (0, 0)),
            pl.BlockSpec((1, H), lambda i: (0, 0)),
        ],
        out_specs=out_spec,
        out_shape=out_shape,
    )(t, s, o, w, b)


def _pool_head_body(h_ref, batch_ref, fcg_ref, fcb_ref, lw_ref, lb_ref,
                    hdg_ref, hdb_ref, cw_ref, cb_ref, hg_ref, out_ref):
    i = pl.program_id(0)

    @pl.when(i == 0)
    def _():
        hg_ref[...] = jnp.zeros_like(hg_ref)
        out_ref[...] = jnp.zeros_like(out_ref)

    bv = batch_ref[...].reshape(1, _BM)
    gi = lax.broadcasted_iota(jnp.int32, (G, _BM), 0)
    onehot = (gi == bv).astype(jnp.float32)
    hg_ref[...] += jnp.dot(onehot, h_ref[...],
                           preferred_element_type=jnp.float32)

    @pl.when(i == _GRID - 1)
    def _():
        hg = hg_ref[...]
        m = jnp.mean(hg, axis=0)
        v = jnp.mean(hg * hg, axis=0) - m * m
        hgn = (hg - m) * lax.rsqrt(v + 1e-5) * fcg_ref[...] + fcb_ref[...]
        h1 = jnp.maximum(
            jnp.dot(hgn, lw_ref[...], preferred_element_type=jnp.float32)
            + lb_ref[...], 0.0)
        m2 = jnp.mean(h1, axis=0)
        v2 = jnp.mean(h1 * h1, axis=0) - m2 * m2
        h1n = (h1 - m2) * lax.rsqrt(v2 + 1e-5) * hdg_ref[...] + hdb_ref[...]
        logits = jnp.dot(h1n, cw_ref[...],
                         preferred_element_type=jnp.float32) + cb_ref[...]
        valid = lax.broadcasted_iota(jnp.int32, (G, 128), 1) < C
        neg = jnp.float32(-1e30)
        lmax = jnp.max(jnp.where(valid, logits, neg), axis=1, keepdims=True)
        shifted = logits - lmax
        se = jnp.sum(jnp.where(valid, jnp.exp(shifted), 0.0), axis=1,
                     keepdims=True)
        out = shifted - jnp.log(se)
        out_ref[...] = jnp.where(valid, out, 0.0)


def _pool_head(h, batch3, fcg, fcb, lw, lb, hdg, hdb, cwp, cbp):
    return pl.pallas_call(
        _pool_head_body,
        grid=(_GRID,),
        in_specs=[
            pl.BlockSpec((_BM, H), lambda i: (i, 0)),
            pl.BlockSpec((1, 1, _BM), lambda i: (i, 0, 0)),
            pl.BlockSpec((1, H), lambda i: (0, 0)),
            pl.BlockSpec((1, H), lambda i: (0, 0)),
            pl.BlockSpec((H, H), lambda i: (0, 0)),
            pl.BlockSpec((1, H), lambda i: (0, 0)),
            pl.BlockSpec((1, H), lambda i: (0, 0)),
            pl.BlockSpec((1, H), lambda i: (0, 0)),
            pl.BlockSpec((H, 128), lambda i: (0, 0)),
            pl.BlockSpec((1, 128), lambda i: (0, 0)),
        ],
        out_specs=[
            pl.BlockSpec((G, H), lambda i: (0, 0)),
            pl.BlockSpec((G, 128), lambda i: (0, 0)),
        ],
        out_shape=[
            jax.ShapeDtypeStruct((G, H), jnp.float32),
            jax.ShapeDtypeStruct((G, 128), jnp.float32),
        ],
    )(h, batch3, fcg, fcb, lw, lb, hdg, hdb, cwp, cbp)


def _scale_offset(st, g, b, n):
    m = st[0] / n
    v = st[1] / n - m * m
    s = g * lax.rsqrt(v + 1e-5)
    o = b - m * s
    return s[None, :], o[None, :]


def kernel(x, edge_index, batch, bn_feat_g, bn_feat_b, W0, b0, gin_W1, gin_b1,
           gin_bn_g, gin_bn_b, gin_W2, gin_b2, bn_fc_g, bn_fc_b, lin_W, lin_b,
           bn_hid_g, bn_hid_b, cls_W, cls_b):
    src = edge_index[0]
    dst = edge_index[1]
    pad = _E_PAD - E
    srcp = jnp.concatenate([src, jnp.zeros((pad,), jnp.int32)])
    dstp = jnp.concatenate([dst, jnp.full((pad,), N, jnp.int32)])
    srcs = jnp.stack([srcp, srcp + N]).reshape(NC, NS, _NB, _CHUNK)
    dst4 = dstp.reshape(NS, _NB, _CHUNK)
    batch3 = batch.reshape(_GRID, 1, _BM)

    st0 = _stats_x(x)
    s0, o0 = _scale_offset(st0, bn_feat_g, bn_feat_b, N)
    h2 = _bn0_mm(x, s0, o0, W0, b0[None, :])

    for i in range(L):
        z2 = _sc_aggregate(h2.reshape(NC * N, FH), srcs, dst4)
        t, st = _mm1_stats(z2, gin_W1[i], gin_b1[i][None, :])
        si, oi = _scale_offset(st, gin_bn_g[i], gin_bn_b[i], N)
        last = i == L - 1
        h2 = _bn_mm2(t, si, oi, gin_W2[i], gin_b2[i][None, :],
                     split_out=not last)

    cwp = jnp.zeros((H, 128), jnp.float32).at[:, :C].set(cls_W)
    cbp = jnp.zeros((1, 128), jnp.float32).at[0, :C].set(cls_b)
    _, logits = _pool_head(h2, batch3, bn_fc_g[None, :], bn_fc_b[None, :],
                           lin_W, lin_b[None, :], bn_hid_g[None, :],
                           bn_hid_b[None, :], cwp, cbp)
    return logits[:, :C]
